# Initial kernel scaffold; baseline (speedup 1.0000x reference)
import jax
import jax.numpy as jnp
from jax.experimental import pallas as pl


def kernel(x, a_indices, a_values, W):
    raise NotImplementedError("write your pallas kernel here")


# speedup vs baseline: 3.1021x; 3.1021x over previous
import dataclasses
import functools

import jax
import jax.numpy as jnp
from jax import lax
from jax.experimental import pallas as pl
from jax.experimental.pallas import tpu as pltpu
from jax.experimental.pallas import tpu_sc as plsc

N_NODES = 10000
N_EDGES = 320000
D = 128

NS = 16
EPW = N_EDGES // NS
CHUNK = 80
BCH = 25
NBLK = EPW // (CHUNK * BCH)
ROWS_PER_SUB = 624


def _matmul(x, W):
    def body(x_ref, w_ref, o_ref):
        o_ref[...] = lax.dot_general(
            x_ref[...], w_ref[...], (((1,), (0,)), ((), ())),
            precision=lax.Precision.HIGHEST,
            preferred_element_type=jnp.float32)

    bm = 2000
    return pl.pallas_call(
        body,
        grid=(N_NODES // bm,),
        in_specs=[
            pl.BlockSpec((bm, D), lambda i: (i, 0)),
            pl.BlockSpec((D, D), lambda i: (0, 0)),
        ],
        out_specs=pl.BlockSpec((bm, D), lambda i: (i, 0)),
        out_shape=jax.ShapeDtypeStruct((N_NODES, D), jnp.float32),
    )(x, W)


def _relu(acc):
    def body(p_ref, o_ref):
        o_ref[...] = jnp.maximum(p_ref[...], 0.0)

    bm = 2000
    return pl.pallas_call(
        body,
        grid=(N_NODES // bm,),
        in_specs=[pl.BlockSpec((bm, D), lambda i: (i, 0))],
        out_specs=pl.BlockSpec((bm, D), lambda i: (i, 0)),
        out_shape=jax.ShapeDtypeStruct((N_NODES, D), jnp.float32),
    )(acc)


_SC_PARAMS = pltpu.CompilerParams()
if "needs_layout_passes" in pltpu.CompilerParams.__dataclass_fields__:
    _SC_PARAMS = dataclasses.replace(_SC_PARAMS, needs_layout_passes=False)


@functools.partial(
    pl.kernel,
    mesh=plsc.VectorSubcoreMesh(core_axis_name="c", subcore_axis_name="s",
                                num_cores=1),
    compiler_params=_SC_PARAMS,
    out_type=jax.ShapeDtypeStruct((N_NODES, D), jnp.float32),
    scratch_types=[
        pltpu.VMEM((BCH, CHUNK), jnp.int32),
        pltpu.VMEM((BCH, CHUNK), jnp.int32),
        pltpu.VMEM((BCH, CHUNK), jnp.float32),
        pltpu.VMEM((CHUNK, D), jnp.float32),
        pltpu.VMEM_SHARED((N_NODES, D), jnp.float32),
        pltpu.SemaphoreType.DMA,
    ],
)
def _sc_scatter(xw_hbm, cols_hbm, rows_hbm, vals_hbm, out_hbm,
                col_v, row_v, val_v, rows_buf, acc_sh, sem):
    sid = lax.axis_index("s")

    @pl.loop(0, CHUNK)
    def _(i):
        for g in range(D // 16):
            rows_buf[i, pl.ds(g * 16, 16)] = jnp.zeros((16,), jnp.float32)

    base = pl.multiple_of(sid * ROWS_PER_SUB, 8)
    for k in range(7):
        pltpu.sync_copy(rows_buf, acc_sh.at[pl.ds(base + k * CHUNK, CHUNK)])
    pltpu.sync_copy(rows_buf.at[pl.ds(0, 64)],
                    acc_sh.at[pl.ds(base + 7 * CHUNK, 64)])

    @pl.when(sid == NS - 1)
    def _():
        pltpu.sync_copy(rows_buf.at[pl.ds(0, 16)],
                        acc_sh.at[pl.ds(NS * ROWS_PER_SUB, 16)])

    plsc.subcore_barrier()

    @pl.loop(0, NBLK)
    def _(b):
        pltpu.sync_copy(cols_hbm.at[sid, b], col_v)
        pltpu.sync_copy(rows_hbm.at[sid, b], row_v)
        pltpu.sync_copy(vals_hbm.at[sid, b], val_v)

        @pl.loop(0, BCH)
        def _(j):
            pltpu.async_copy(xw_hbm.at[col_v.at[j]], rows_buf, sem).wait()

            @pl.loop(0, CHUNK)
            def _(e):
                bval = plsc.load_gather(
                    val_v,
                    [jnp.full((16,), j, jnp.int32),
                     jnp.full((16,), e, jnp.int32)])
                for g in range(D // 16):
                    sl = (e, pl.ds(g * 16, 16))
                    rows_buf[sl] = rows_buf[sl] * bval

            pltpu.sync_copy(rows_buf, acc_sh.at[row_v.at[j]], add=True)

    plsc.subcore_barrier()
    pltpu.sync_copy(acc_sh.at[pl.ds(base, ROWS_PER_SUB)],
                    out_hbm.at[pl.ds(base, ROWS_PER_SUB)])

    @pl.when(sid == NS - 1)
    def _():
        pltpu.sync_copy(acc_sh.at[pl.ds(NS * ROWS_PER_SUB, 16)],
                        out_hbm.at[pl.ds(NS * ROWS_PER_SUB, 16)])


def kernel(x, a_indices, a_values, W):
    xw = _matmul(x, W)
    rows = a_indices[0].reshape(NS, NBLK, BCH, CHUNK)
    cols = a_indices[1].reshape(NS, NBLK, BCH, CHUNK)
    vals = a_values.reshape(NS, NBLK, BCH, CHUNK)
    acc = _sc_scatter(xw, cols, rows, vals)
    return _relu(acc)


# speedup vs baseline: 5.3005x; 1.7087x over previous
import dataclasses
import functools

import jax
import jax.numpy as jnp
from jax import lax
from jax.experimental import pallas as pl
from jax.experimental.pallas import tpu as pltpu
from jax.experimental.pallas import tpu_sc as plsc

N_NODES = 10000
N_EDGES = 320000
D = 128

NS = 16
EPW = N_EDGES // NS
CHUNK = 100
BCH = 50
NBLK = EPW // (CHUNK * BCH)
ROWS_PER_SUB = 624


def _matmul(x, W):
    def body(x_ref, w_ref, o_ref):
        o_ref[...] = lax.dot_general(
            x_ref[...], w_ref[...], (((1,), (0,)), ((), ())),
            precision=lax.Precision.HIGHEST,
            preferred_element_type=jnp.float32)

    bm = 2000
    return pl.pallas_call(
        body,
        grid=(N_NODES // bm,),
        in_specs=[
            pl.BlockSpec((bm, D), lambda i: (i, 0)),
            pl.BlockSpec((D, D), lambda i: (0, 0)),
        ],
        out_specs=pl.BlockSpec((bm, D), lambda i: (i, 0)),
        out_shape=jax.ShapeDtypeStruct((N_NODES, D), jnp.float32),
    )(x, W)


def _relu(acc):
    def body(p_ref, o_ref):
        o_ref[...] = jnp.maximum(p_ref[...], 0.0)

    bm = 2000
    return pl.pallas_call(
        body,
        grid=(N_NODES // bm,),
        in_specs=[pl.BlockSpec((bm, D), lambda i: (i, 0))],
        out_specs=pl.BlockSpec((bm, D), lambda i: (i, 0)),
        out_shape=jax.ShapeDtypeStruct((N_NODES, D), jnp.float32),
    )(acc)


_SC_PARAMS = pltpu.CompilerParams()
if "needs_layout_passes" in pltpu.CompilerParams.__dataclass_fields__:
    _SC_PARAMS = dataclasses.replace(_SC_PARAMS, needs_layout_passes=False)


@functools.partial(
    pl.kernel,
    mesh=plsc.VectorSubcoreMesh(core_axis_name="c", subcore_axis_name="s",
                                num_cores=1),
    compiler_params=_SC_PARAMS,
    out_type=jax.ShapeDtypeStruct((N_NODES, D), jnp.float32),
    scratch_types=[
        pltpu.VMEM((BCH, CHUNK), jnp.int32),
        pltpu.VMEM((BCH, CHUNK), jnp.int32),
        pltpu.VMEM((BCH, CHUNK), jnp.float32),
        pltpu.VMEM((CHUNK, D), jnp.float32),
        pltpu.VMEM((CHUNK, D), jnp.float32),
        pltpu.VMEM_SHARED((N_NODES, D), jnp.float32),
        pltpu.SemaphoreType.DMA,
        pltpu.SemaphoreType.DMA,
    ],
)
def _sc_scatter(xw_hbm, cols_hbm, rows_hbm, vals_hbm, out_hbm,
                col_v, row_v, val_v, rows_a, rows_b, acc_sh, sem_a, sem_b):
    sid = lax.axis_index("s")

    @pl.loop(0, CHUNK)
    def _(i):
        for g in range(D // 16):
            rows_a[i, pl.ds(g * 16, 16)] = jnp.zeros((16,), jnp.float32)

    base = pl.multiple_of(sid * ROWS_PER_SUB, 8)
    for k in range(6):
        pltpu.sync_copy(rows_a, acc_sh.at[pl.ds(base + k * CHUNK, CHUNK)])
    pltpu.sync_copy(rows_a.at[pl.ds(0, 24)],
                    acc_sh.at[pl.ds(base + 6 * CHUNK, 24)])

    @pl.when(sid == NS - 1)
    def _():
        pltpu.sync_copy(rows_a.at[pl.ds(0, 16)],
                        acc_sh.at[pl.ds(NS * ROWS_PER_SUB, 16)])

    plsc.subcore_barrier()

    def g_issue(j, buf, sem):
        pltpu.async_copy(xw_hbm.at[col_v.at[j]], buf, sem)

    def g_wait(buf, sem):
        pltpu.make_async_copy(xw_hbm.at[col_v.at[0]], buf, sem).wait()

    def scale(j, buf):
        jv = jnp.full((16,), j, jnp.int32)

        @pl.loop(0, CHUNK)
        def _(e):
            bval = plsc.load_gather(val_v, [jv, jnp.full((16,), e, jnp.int32)])
            for g in range(D // 16):
                sl = (e, pl.ds(g * 16, 16))
                buf[sl] = buf[sl] * bval

    def s_sync(j, buf):
        pltpu.sync_copy(buf, acc_sh.at[row_v.at[j]], add=True)

    @pl.loop(0, NBLK)
    def _(b):
        pltpu.sync_copy(cols_hbm.at[sid, b], col_v)
        pltpu.sync_copy(rows_hbm.at[sid, b], row_v)
        pltpu.sync_copy(vals_hbm.at[sid, b], val_v)

        g_issue(0, rows_a, sem_a)

        @pl.loop(0, BCH // 2 - 1)
        def _(k):
            j0 = 2 * k
            g_wait(rows_a, sem_a)
            g_issue(j0 + 1, rows_b, sem_b)
            scale(j0, rows_a)
            s_sync(j0, rows_a)
            g_wait(rows_b, sem_b)
            g_issue(j0 + 2, rows_a, sem_a)
            scale(j0 + 1, rows_b)
            s_sync(j0 + 1, rows_b)

        g_wait(rows_a, sem_a)
        g_issue(BCH - 1, rows_b, sem_b)
        scale(BCH - 2, rows_a)
        s_sync(BCH - 2, rows_a)
        g_wait(rows_b, sem_b)
        scale(BCH - 1, rows_b)
        s_sync(BCH - 1, rows_b)

    plsc.subcore_barrier()
    pltpu.sync_copy(acc_sh.at[pl.ds(base, ROWS_PER_SUB)],
                    out_hbm.at[pl.ds(base, ROWS_PER_SUB)])

    @pl.when(sid == NS - 1)
    def _():
        pltpu.sync_copy(acc_sh.at[pl.ds(NS * ROWS_PER_SUB, 16)],
                        out_hbm.at[pl.ds(NS * ROWS_PER_SUB, 16)])


def kernel(x, a_indices, a_values, W):
    xw = _matmul(x, W)
    rows = a_indices[0].reshape(NS, NBLK, BCH, CHUNK)
    cols = a_indices[1].reshape(NS, NBLK, BCH, CHUNK)
    vals = a_values.reshape(NS, NBLK, BCH, CHUNK)
    acc = _sc_scatter(xw, cols, rows, vals)
    return _relu(acc)


# speedup vs baseline: 6.5822x; 1.2418x over previous
import dataclasses
import functools

import jax
import jax.numpy as jnp
from jax import lax
from jax.experimental import pallas as pl
from jax.experimental.pallas import tpu as pltpu
from jax.experimental.pallas import tpu_sc as plsc

N_NODES = 10000
N_EDGES = 320000
D = 128

NS = 16
EPW = N_EDGES // NS
CHUNK = 125
BCH = 32
NBLK = EPW // (CHUNK * BCH)
ROWS_PER_SUB = 624


def _matmul(x, W):
    def body(x_ref, w_ref, o_ref):
        o_ref[...] = lax.dot_general(
            x_ref[...], w_ref[...], (((1,), (0,)), ((), ())),
            precision=lax.Precision.HIGHEST,
            preferred_element_type=jnp.float32)

    bm = 2000
    return pl.pallas_call(
        body,
        grid=(N_NODES // bm,),
        in_specs=[
            pl.BlockSpec((bm, D), lambda i: (i, 0)),
            pl.BlockSpec((D, D), lambda i: (0, 0)),
        ],
        out_specs=pl.BlockSpec((bm, D), lambda i: (i, 0)),
        out_shape=jax.ShapeDtypeStruct((N_NODES, D), jnp.float32),
    )(x, W)


_SC_PARAMS = pltpu.CompilerParams()
if "needs_layout_passes" in pltpu.CompilerParams.__dataclass_fields__:
    _SC_PARAMS = dataclasses.replace(_SC_PARAMS, needs_layout_passes=False)


@functools.partial(
    pl.kernel,
    mesh=plsc.VectorSubcoreMesh(core_axis_name="c", subcore_axis_name="s",
                                num_cores=1),
    compiler_params=_SC_PARAMS,
    out_type=jax.ShapeDtypeStruct((N_NODES, D), jnp.float32),
    scratch_types=[
        pltpu.VMEM((BCH, CHUNK), jnp.int32),
        pltpu.VMEM((BCH, CHUNK), jnp.int32),
        pltpu.VMEM((BCH, CHUNK), jnp.float32),
        pltpu.VMEM((CHUNK, D), jnp.float32),
        pltpu.VMEM((CHUNK, D), jnp.float32),
        pltpu.VMEM_SHARED((N_NODES, D), jnp.float32),
        pltpu.SemaphoreType.DMA,
        pltpu.SemaphoreType.DMA,
        pltpu.SemaphoreType.DMA,
        pltpu.SemaphoreType.DMA,
    ],
)
def _sc_scatter(xw_hbm, cols_hbm, rows_hbm, vals_hbm, out_hbm,
                col_v, row_v, val_v, rows_a, rows_b, acc_sh,
                sem_ga, sem_gb, sem_sa, sem_sb):
    sid = lax.axis_index("s")

    @pl.loop(0, CHUNK)
    def _(i):
        for g in range(D // 16):
            rows_a[i, pl.ds(g * 16, 16)] = jnp.zeros((16,), jnp.float32)

    base = pl.multiple_of(sid * ROWS_PER_SUB, 8)
    for k in range(5):
        pltpu.sync_copy(rows_a.at[pl.ds(0, 120)],
                        acc_sh.at[pl.ds(base + k * 120, 120)])
    pltpu.sync_copy(rows_a.at[pl.ds(0, 24)],
                    acc_sh.at[pl.ds(base + 600, 24)])

    @pl.when(sid == NS - 1)
    def _():
        pltpu.sync_copy(rows_a.at[pl.ds(0, 16)],
                        acc_sh.at[pl.ds(NS * ROWS_PER_SUB, 16)])

    plsc.subcore_barrier()

    def g_issue(j, buf, sem):
        pltpu.async_copy(xw_hbm.at[col_v.at[j]], buf, sem)

    def g_wait(buf, sem):
        pltpu.make_async_copy(xw_hbm.at[col_v.at[0]], buf, sem).wait()

    def scale(j, buf):
        jv = jnp.full((16,), j, jnp.int32)

        @plsc.parallel_loop(0, CHUNK, unroll=5)
        def _(e):
            bval = plsc.load_gather(val_v, [jv, jnp.full((16,), e, jnp.int32)])
            for g in range(D // 16):
                sl = (e, pl.ds(g * 16, 16))
                buf[sl] = buf[sl] * bval

    def s_issue(j, buf, sem):
        pltpu.async_copy(buf, acc_sh.at[row_v.at[j]], sem, add=True)

    def s_wait(buf, sem):
        pltpu.make_async_copy(buf, acc_sh.at[row_v.at[0]], sem).wait()

    @pl.loop(0, NBLK)
    def _(b):
        pltpu.sync_copy(cols_hbm.at[sid, b], col_v)
        pltpu.sync_copy(rows_hbm.at[sid, b], row_v)
        pltpu.sync_copy(vals_hbm.at[sid, b], val_v)

        g_issue(0, rows_a, sem_ga)
        g_wait(rows_a, sem_ga)
        g_issue(1, rows_b, sem_gb)
        scale(0, rows_a)
        s_issue(0, rows_a, sem_sa)

        @pl.loop(1, BCH - 1, step=2)
        def _(j):
            g_wait(rows_b, sem_gb)
            s_wait(rows_a, sem_sa)
            g_issue(j + 1, rows_a, sem_ga)
            scale(j, rows_b)
            s_issue(j, rows_b, sem_sb)
            g_wait(rows_a, sem_ga)
            s_wait(rows_b, sem_sb)
            g_issue(j + 2, rows_b, sem_gb)
            scale(j + 1, rows_a)
            s_issue(j + 1, rows_a, sem_sa)

        g_wait(rows_b, sem_gb)
        s_wait(rows_a, sem_sa)
        scale(BCH - 1, rows_b)
        s_issue(BCH - 1, rows_b, sem_sb)
        s_wait(rows_b, sem_sb)

    plsc.subcore_barrier()

    def relu_out(off, nrows):
        pltpu.sync_copy(acc_sh.at[pl.ds(off, nrows)],
                        rows_a.at[pl.ds(0, nrows)])

        @pl.loop(0, nrows)
        def _(i):
            for g in range(D // 16):
                sl = (i, pl.ds(g * 16, 16))
                rows_a[sl] = jnp.maximum(rows_a[sl], 0.0)

        pltpu.sync_copy(rows_a.at[pl.ds(0, nrows)],
                        out_hbm.at[pl.ds(off, nrows)])

    for k in range(5):
        relu_out(base + k * 120, 120)
    relu_out(base + 600, 24)

    @pl.when(sid == NS - 1)
    def _():
        relu_out(NS * ROWS_PER_SUB, 16)


def kernel(x, a_indices, a_values, W):
    xw = _matmul(x, W)
    rows = a_indices[0].reshape(NS, NBLK, BCH, CHUNK)
    cols = a_indices[1].reshape(NS, NBLK, BCH, CHUNK)
    vals = a_values.reshape(NS, NBLK, BCH, CHUNK)
    return _sc_scatter(xw, cols, rows, vals)


# speedup vs baseline: 6.8509x; 1.0408x over previous
import dataclasses
import functools

import jax
import jax.numpy as jnp
from jax import lax
from jax.experimental import pallas as pl
from jax.experimental.pallas import tpu as pltpu
from jax.experimental.pallas import tpu_sc as plsc

N_NODES = 10000
N_EDGES = 320000
D = 128

NS = 16
EPW = N_EDGES // NS
CHUNK = 100
BCH = 20
NBLK = EPW // (CHUNK * BCH)
ROWS_PER_SUB = 624


def _matmul(x, W):
    def body(x_ref, w_ref, o_ref):
        o_ref[...] = lax.dot_general(
            x_ref[...], w_ref[...], (((1,), (0,)), ((), ())),
            precision=lax.Precision.HIGHEST,
            preferred_element_type=jnp.float32)

    bm = 2000
    return pl.pallas_call(
        body,
        grid=(N_NODES // bm,),
        in_specs=[
            pl.BlockSpec((bm, D), lambda i: (i, 0)),
            pl.BlockSpec((D, D), lambda i: (0, 0)),
        ],
        out_specs=pl.BlockSpec((bm, D), lambda i: (i, 0)),
        out_shape=jax.ShapeDtypeStruct((N_NODES, D), jnp.float32),
    )(x, W)


_SC_PARAMS = pltpu.CompilerParams()
if "needs_layout_passes" in pltpu.CompilerParams.__dataclass_fields__:
    _SC_PARAMS = dataclasses.replace(_SC_PARAMS, needs_layout_passes=False)


@functools.partial(
    pl.kernel,
    mesh=plsc.VectorSubcoreMesh(core_axis_name="c", subcore_axis_name="s",
                                num_cores=1),
    compiler_params=_SC_PARAMS,
    out_type=jax.ShapeDtypeStruct((N_NODES, D), jnp.float32),
    scratch_types=[
        pltpu.VMEM((BCH, CHUNK), jnp.int32),
        pltpu.VMEM((BCH, CHUNK), jnp.int32),
        pltpu.VMEM((BCH, CHUNK), jnp.float32),
        pltpu.VMEM((CHUNK, D), jnp.float32),
        pltpu.VMEM((CHUNK, D), jnp.float32),
        pltpu.VMEM((CHUNK, D), jnp.float32),
        pltpu.VMEM_SHARED((N_NODES, D), jnp.float32),
        pltpu.SemaphoreType.DMA,
        pltpu.SemaphoreType.DMA,
        pltpu.SemaphoreType.DMA,
        pltpu.SemaphoreType.DMA,
        pltpu.SemaphoreType.DMA,
        pltpu.SemaphoreType.DMA,
    ],
)
def _sc_scatter(xw_hbm, cols_hbm, rows_hbm, vals_hbm, out_hbm,
                col_v, row_v, val_v, rows_0, rows_1, rows_2, acc_sh,
                sem_g0, sem_g1, sem_g2, sem_s0, sem_s1, sem_s2):
    sid = lax.axis_index("s")
    bufs = (rows_0, rows_1, rows_2)
    gsems = (sem_g0, sem_g1, sem_g2)
    ssems = (sem_s0, sem_s1, sem_s2)

    @pl.loop(0, 96)
    def _(i):
        for g in range(D // 16):
            rows_0[i, pl.ds(g * 16, 16)] = jnp.zeros((16,), jnp.float32)

    base = pl.multiple_of(sid * ROWS_PER_SUB, 8)
    for k in range(6):
        pltpu.sync_copy(rows_0.at[pl.ds(0, 96)],
                        acc_sh.at[pl.ds(base + k * 96, 96)])
    pltpu.sync_copy(rows_0.at[pl.ds(0, 48)],
                    acc_sh.at[pl.ds(base + 576, 48)])

    @pl.when(sid == NS - 1)
    def _():
        pltpu.sync_copy(rows_0.at[pl.ds(0, 16)],
                        acc_sh.at[pl.ds(NS * ROWS_PER_SUB, 16)])

    plsc.subcore_barrier()

    def g_issue(j, buf, sem):
        pltpu.async_copy(xw_hbm.at[col_v.at[j]], buf, sem)

    def g_wait(buf, sem):
        pltpu.make_async_copy(xw_hbm.at[col_v.at[0]], buf, sem).wait()

    def scale(j, buf):
        jv = jnp.full((16,), j, jnp.int32)

        @plsc.parallel_loop(0, CHUNK, unroll=5)
        def _(e):
            bval = plsc.load_gather(val_v, [jv, jnp.full((16,), e, jnp.int32)])
            for g in range(D // 16):
                sl = (e, pl.ds(g * 16, 16))
                buf[sl] = buf[sl] * bval

    def s_issue(j, buf, sem):
        pltpu.async_copy(buf, acc_sh.at[row_v.at[j]], sem, add=True)

    def s_wait(buf, sem):
        pltpu.make_async_copy(buf, acc_sh.at[row_v.at[0]], sem).wait()

    @pl.loop(0, NBLK)
    def _(b):
        pltpu.sync_copy(cols_hbm.at[sid, b], col_v)
        pltpu.sync_copy(rows_hbm.at[sid, b], row_v)
        pltpu.sync_copy(vals_hbm.at[sid, b], val_v)

        g_issue(0, bufs[0], gsems[0])
        g_issue(1, bufs[1], gsems[1])
        for c in range(BCH):
            i = c % 3
            g_wait(bufs[i], gsems[i])
            scale(c, bufs[i])
            s_issue(c, bufs[i], ssems[i])
            if c >= 1:
                ip = (c - 1) % 3
                s_wait(bufs[ip], ssems[ip])
            if c + 2 < BCH:
                inx = (c + 2) % 3
                g_issue(c + 2, bufs[inx], gsems[inx])
        ilast = (BCH - 1) % 3
        s_wait(bufs[ilast], ssems[ilast])

    plsc.subcore_barrier()

    def relu_out(off, nrows):
        pltpu.sync_copy(acc_sh.at[pl.ds(off, nrows)],
                        rows_0.at[pl.ds(0, nrows)])

        @pl.loop(0, nrows)
        def _(i):
            for g in range(D // 16):
                sl = (i, pl.ds(g * 16, 16))
                rows_0[sl] = jnp.maximum(rows_0[sl], 0.0)

        pltpu.sync_copy(rows_0.at[pl.ds(0, nrows)],
                        out_hbm.at[pl.ds(off, nrows)])

    for k in range(6):
        relu_out(base + k * 96, 96)
    relu_out(base + 576, 48)

    @pl.when(sid == NS - 1)
    def _():
        relu_out(NS * ROWS_PER_SUB, 16)


def kernel(x, a_indices, a_values, W):
    xw = _matmul(x, W)
    rows = a_indices[0].reshape(NS, NBLK, BCH, CHUNK)
    cols = a_indices[1].reshape(NS, NBLK, BCH, CHUNK)
    vals = a_values.reshape(NS, NBLK, BCH, CHUNK)
    return _sc_scatter(xw, cols, rows, vals)


# speedup vs baseline: 6.9028x; 1.0076x over previous
import dataclasses
import functools

import jax
import jax.numpy as jnp
from jax import lax
from jax.experimental import pallas as pl
from jax.experimental.pallas import tpu as pltpu
from jax.experimental.pallas import tpu_sc as plsc

N_NODES = 10000
N_EDGES = 320000
D = 128

NS = 16
EPW = N_EDGES // NS
CHUNK = 80
BCH = 25
NBLK = EPW // (CHUNK * BCH)
ROWS_PER_SUB = 624


def _matmul(x, W):
    def body(x_ref, w_ref, o_ref):
        o_ref[...] = lax.dot_general(
            x_ref[...], w_ref[...], (((1,), (0,)), ((), ())),
            precision=lax.Precision.HIGHEST,
            preferred_element_type=jnp.float32)

    bm = 2000
    return pl.pallas_call(
        body,
        grid=(N_NODES // bm,),
        in_specs=[
            pl.BlockSpec((bm, D), lambda i: (i, 0)),
            pl.BlockSpec((D, D), lambda i: (0, 0)),
        ],
        out_specs=pl.BlockSpec((bm, D), lambda i: (i, 0)),
        out_shape=jax.ShapeDtypeStruct((N_NODES, D), jnp.float32),
    )(x, W)


_SC_PARAMS = pltpu.CompilerParams()
if "needs_layout_passes" in pltpu.CompilerParams.__dataclass_fields__:
    _SC_PARAMS = dataclasses.replace(_SC_PARAMS, needs_layout_passes=False)


@functools.partial(
    pl.kernel,
    mesh=plsc.VectorSubcoreMesh(core_axis_name="c", subcore_axis_name="s",
                                num_cores=1),
    compiler_params=_SC_PARAMS,
    out_type=jax.ShapeDtypeStruct((N_NODES, D), jnp.float32),
    scratch_types=[
        pltpu.VMEM((BCH, CHUNK), jnp.int32),
        pltpu.VMEM((BCH, CHUNK), jnp.int32),
        pltpu.VMEM((BCH, CHUNK), jnp.float32),
        pltpu.VMEM((CHUNK, D), jnp.float32),
        pltpu.VMEM((CHUNK, D), jnp.float32),
        pltpu.VMEM((CHUNK, D), jnp.float32),
        pltpu.VMEM_SHARED((N_NODES, D), jnp.float32),
        pltpu.SemaphoreType.DMA,
        pltpu.SemaphoreType.DMA,
        pltpu.SemaphoreType.DMA,
        pltpu.SemaphoreType.DMA,
        pltpu.SemaphoreType.DMA,
        pltpu.SemaphoreType.DMA,
    ],
)
def _sc_scatter(xw_hbm, cols_hbm, rows_hbm, vals_hbm, out_hbm,
                col_v, row_v, val_v, rows_0, rows_1, rows_2, acc_sh,
                sem_g0, sem_g1, sem_g2, sem_s0, sem_s1, sem_s2):
    sid = lax.axis_index("s")
    bufs = (rows_0, rows_1, rows_2)
    gsems = (sem_g0, sem_g1, sem_g2)
    ssems = (sem_s0, sem_s1, sem_s2)

    @pl.loop(0, 80)
    def _(i):
        for g in range(D // 16):
            rows_0[i, pl.ds(g * 16, 16)] = jnp.zeros((16,), jnp.float32)

    base = pl.multiple_of(sid * ROWS_PER_SUB, 8)
    for k in range(7):
        pltpu.async_copy(rows_0.at[pl.ds(0, 80)],
                         acc_sh.at[pl.ds(base + k * 80, 80)], sem_s0)
    pltpu.async_copy(rows_0.at[pl.ds(0, 64)],
                     acc_sh.at[pl.ds(base + 560, 64)], sem_s0)

    @pl.when(sid == NS - 1)
    def _():
        pltpu.async_copy(rows_0.at[pl.ds(0, 16)],
                         acc_sh.at[pl.ds(NS * ROWS_PER_SUB, 16)], sem_s0)

    for k in range(7):
        pltpu.make_async_copy(rows_0.at[pl.ds(0, 80)],
                              acc_sh.at[pl.ds(base + k * 80, 80)],
                              sem_s0).wait()
    pltpu.make_async_copy(rows_0.at[pl.ds(0, 64)],
                          acc_sh.at[pl.ds(base + 560, 64)], sem_s0).wait()

    @pl.when(sid == NS - 1)
    def _():
        pltpu.make_async_copy(rows_0.at[pl.ds(0, 16)],
                              acc_sh.at[pl.ds(NS * ROWS_PER_SUB, 16)],
                              sem_s0).wait()

    plsc.subcore_barrier()

    def g_issue(j, buf, sem):
        pltpu.async_copy(xw_hbm.at[col_v.at[j]], buf, sem)

    def g_wait(buf, sem):
        pltpu.make_async_copy(xw_hbm.at[col_v.at[0]], buf, sem).wait()

    def scale(j, buf):
        @plsc.parallel_loop(0, CHUNK, step=16)
        def _(e0):
            val16 = val_v[j, pl.ds(e0, 16)]
            for l in range(16):
                bval = val16[l]
                for g in range(D // 16):
                    sl = (e0 + l, pl.ds(g * 16, 16))
                    buf[sl] = buf[sl] * bval

    def s_issue(j, buf, sem):
        pltpu.async_copy(buf, acc_sh.at[row_v.at[j]], sem, add=True)

    def s_wait(buf, sem):
        pltpu.make_async_copy(buf, acc_sh.at[row_v.at[0]], sem).wait()

    @pl.loop(0, NBLK)
    def _(b):
        pltpu.sync_copy(cols_hbm.at[sid, b], col_v)
        pltpu.sync_copy(rows_hbm.at[sid, b], row_v)
        pltpu.sync_copy(vals_hbm.at[sid, b], val_v)

        g_issue(0, bufs[0], gsems[0])
        g_issue(1, bufs[1], gsems[1])
        g_wait(bufs[0], gsems[0])
        scale(0, bufs[0])
        s_issue(0, bufs[0], ssems[0])
        g_issue(2, bufs[2], gsems[2])

        @pl.loop(0, (BCH - 1) // 3)
        def _(t):
            c0 = 3 * t + 1

            g_wait(bufs[1], gsems[1])
            scale(c0, bufs[1])
            s_issue(c0, bufs[1], ssems[1])
            s_wait(bufs[0], ssems[0])
            g_issue(c0 + 2, bufs[0], gsems[0])

            g_wait(bufs[2], gsems[2])
            scale(c0 + 1, bufs[2])
            s_issue(c0 + 1, bufs[2], ssems[2])
            s_wait(bufs[1], ssems[1])

            @pl.when(c0 + 3 < BCH)
            def _():
                g_issue(c0 + 3, bufs[1], gsems[1])

            g_wait(bufs[0], gsems[0])
            scale(c0 + 2, bufs[0])
            s_issue(c0 + 2, bufs[0], ssems[0])
            s_wait(bufs[2], ssems[2])

            @pl.when(c0 + 4 < BCH)
            def _():
                g_issue(c0 + 4, bufs[2], gsems[2])

        s_wait(bufs[0], ssems[0])

    plsc.subcore_barrier()

    def relu_buf(b, n):
        @pl.loop(0, n)
        def _(i):
            for g in range(D // 16):
                sl = (i, pl.ds(g * 16, 16))
                b[sl] = jnp.maximum(b[sl], 0.0)

    rc = [(base + k * 80, 80) for k in range(7)] + [(base + 560, 64)]

    def rin(k, issue):
        off, n = rc[k]
        cp = (pltpu.async_copy if issue else pltpu.make_async_copy)(
            acc_sh.at[pl.ds(off, n)], bufs[k % 2].at[pl.ds(0, n)],
            gsems[k % 2])
        if not issue:
            cp.wait()

    def rout(k, issue):
        off, n = rc[k]
        cp = (pltpu.async_copy if issue else pltpu.make_async_copy)(
            bufs[k % 2].at[pl.ds(0, n)], out_hbm.at[pl.ds(off, n)],
            ssems[k % 2])
        if not issue:
            cp.wait()

    rin(0, True)
    for k in range(len(rc)):
        rin(k, False)
        if k + 1 < len(rc):
            if k >= 1:
                rout(k - 1, False)
            rin(k + 1, True)
        relu_buf(bufs[k % 2], rc[k][1])
        rout(k, True)
    rout(len(rc) - 2, False)
    rout(len(rc) - 1, False)

    @pl.when(sid == NS - 1)
    def _():
        pltpu.sync_copy(acc_sh.at[pl.ds(NS * ROWS_PER_SUB, 16)],
                        rows_2.at[pl.ds(0, 16)])
        relu_buf(rows_2, 16)
        pltpu.sync_copy(rows_2.at[pl.ds(0, 16)],
                        out_hbm.at[pl.ds(NS * ROWS_PER_SUB, 16)])


def kernel(x, a_indices, a_values, W):
    xw = _matmul(x, W)
    rows = a_indices[0].reshape(NS, NBLK, BCH, CHUNK)
    cols = a_indices[1].reshape(NS, NBLK, BCH, CHUNK)
    vals = a_values.reshape(NS, NBLK, BCH, CHUNK)
    return _sc_scatter(xw, cols, rows, vals)
